# Initial kernel scaffold; baseline (speedup 1.0000x reference)
import jax
import jax.numpy as jnp
from jax.experimental import pallas as pl


def kernel(x, edge_index, W1, b1, W2, b2):
    raise NotImplementedError("write your pallas kernel here")


# speedup vs baseline: 16.0655x; 16.0655x over previous
import functools

import jax
import jax.numpy as jnp
from jax import lax
from jax.experimental import pallas as pl
from jax.experimental.pallas import tpu as pltpu
from jax.experimental.pallas import tpu_sc as plsc

NC, NS = 2, 16
NW = NC * NS
K = 80


def _sc_mesh():
    return plsc.VectorSubcoreMesh(
        core_axis_name="c", subcore_axis_name="s", num_cores=NC, num_subcores=NS)


def _make_deg(n, e):
    ept = e // NW
    assert e % NW == 0 and ept % K == 0

    def body(dst_hbm, zero_hbm, out_hbm, didx, ones, acc, sem):
        c = lax.axis_index("c")
        s = lax.axis_index("s")

        @pl.when(s == 0)
        def _():
            pltpu.sync_copy(zero_hbm, acc)

        for j in range(K // 16):
            ones[pl.ds(j * 16, 16)] = jnp.full((16,), 1.0, jnp.float32)
        plsc.subcore_barrier()

        base = (c * NS + s) * ept

        def step(i, carry):
            pltpu.sync_copy(dst_hbm.at[pl.ds(base + i * K, K)], didx)
            pltpu.sync_copy(ones, acc.at[didx], add=True)
            return carry

        lax.fori_loop(0, ept // K, step, 0)
        plsc.subcore_barrier()

        @pl.when(s == 0)
        def _():
            pltpu.sync_copy(acc, out_hbm.at[c])

    return pl.kernel(
        body,
        out_type=jax.ShapeDtypeStruct((NC, n), jnp.float32),
        mesh=_sc_mesh(),
        compiler_params=pltpu.CompilerParams(use_tc_tiling_on_sc=False),
        scratch_types=[
            pltpu.VMEM((K,), jnp.int32),
            pltpu.VMEM((K,), jnp.float32),
            pltpu.VMEM_SHARED((n,), jnp.float32),
            pltpu.SemaphoreType.DMA,
        ],
    )


def _make_scatter(n, e, f):
    ept = e // NW
    assert e % NW == 0 and ept % K == 0

    def body(y_hbm, src_hbm, dst_hbm, zero_hbm, out_hbm,
             sidx, didx, rows, acc, sem):
        c = lax.axis_index("c")
        s = lax.axis_index("s")

        @pl.when(s == 0)
        def _():
            pltpu.sync_copy(zero_hbm, acc)
        plsc.subcore_barrier()

        base = (c * NS + s) * ept

        def step(i, carry):
            eb = base + i * K
            pltpu.sync_copy(src_hbm.at[pl.ds(eb, K)], sidx)
            pltpu.sync_copy(dst_hbm.at[pl.ds(eb, K)], didx)
            pltpu.async_copy(y_hbm.at[sidx], rows, sem).wait()
            pltpu.sync_copy(rows, acc.at[didx], add=True)
            return carry

        lax.fori_loop(0, ept // K, step, 0)
        plsc.subcore_barrier()

        @pl.when(s == 0)
        def _():
            pltpu.sync_copy(acc, out_hbm.at[c])

    return pl.kernel(
        body,
        out_type=jax.ShapeDtypeStruct((NC, n, f), jnp.float32),
        mesh=_sc_mesh(),
        compiler_params=pltpu.CompilerParams(use_tc_tiling_on_sc=False),
        scratch_types=[
            pltpu.VMEM((K,), jnp.int32),
            pltpu.VMEM((K,), jnp.int32),
            pltpu.VMEM((K, f), jnp.float32),
            pltpu.VMEM_SHARED((n, f), jnp.float32),
            pltpu.SemaphoreType.DMA,
        ],
    )


def _tc_a_body(dcol_ref, x_ref, w1_ref, y1_ref):
    xw = jnp.dot(x_ref[...], w1_ref[...], preferred_element_type=jnp.float32)
    y1_ref[...] = xw * dcol_ref[...]


def _tc_b_body(p1a_ref, p1b_ref, y1_ref, dcol_ref, b1_ref, w2_ref, y2_ref):
    d = dcol_ref[...]
    h = (p1a_ref[...] + p1b_ref[...] + y1_ref[...]) * d + b1_ref[...][None, :]
    h = jnp.maximum(h, 0.0)
    y2_ref[...] = jnp.dot(h, w2_ref[...], preferred_element_type=jnp.float32) * d


def _tc_c_body(p2a_ref, p2b_ref, y2_ref, dcol_ref, b2_ref, out_ref):
    o = (p2a_ref[...] + p2b_ref[...] + y2_ref[...]) * dcol_ref[...] \
        + b2_ref[...][None, :]
    m = jnp.max(o, axis=1, keepdims=True)
    ex = jnp.exp(o - m)
    out_ref[...] = o - m - jnp.log(jnp.sum(ex, axis=1, keepdims=True))


def _row_grid(n, r):
    assert n % r == 0
    return n // r


def _tc_a(dcol, x, w1, r=2000):
    n, dft = x.shape
    h = w1.shape[1]
    return pl.pallas_call(
        _tc_a_body,
        grid=(_row_grid(n, r),),
        in_specs=[
            pl.BlockSpec((r, 1), lambda g: (g, 0)),
            pl.BlockSpec((r, dft), lambda g: (g, 0)),
            pl.BlockSpec((dft, h), lambda g: (0, 0)),
        ],
        out_specs=pl.BlockSpec((r, h), lambda g: (g, 0)),
        out_shape=jax.ShapeDtypeStruct((n, h), jnp.float32),
    )(dcol, x, w1)


def _tc_b(p1a, p1b, y1, dcol, b1, w2, r=2000):
    n, h = y1.shape
    c = w2.shape[1]
    row = lambda g: (g, 0)
    return pl.pallas_call(
        _tc_b_body,
        grid=(_row_grid(n, r),),
        in_specs=[
            pl.BlockSpec((r, h), row),
            pl.BlockSpec((r, h), row),
            pl.BlockSpec((r, h), row),
            pl.BlockSpec((r, 1), row),
            pl.BlockSpec((h,), lambda g: (0,)),
            pl.BlockSpec((h, c), lambda g: (0, 0)),
        ],
        out_specs=pl.BlockSpec((r, c), row),
        out_shape=jax.ShapeDtypeStruct((n, c), jnp.float32),
    )(p1a, p1b, y1, dcol, b1, w2)


def _tc_c(p2a, p2b, y2, dcol, b2, r=2000):
    n, c = y2.shape
    row = lambda g: (g, 0)
    return pl.pallas_call(
        _tc_c_body,
        grid=(_row_grid(n, r),),
        in_specs=[
            pl.BlockSpec((r, c), row),
            pl.BlockSpec((r, c), row),
            pl.BlockSpec((r, c), row),
            pl.BlockSpec((r, 1), row),
            pl.BlockSpec((c,), lambda g: (0,)),
        ],
        out_specs=pl.BlockSpec((r, c), row),
        out_shape=jax.ShapeDtypeStruct((n, c), jnp.float32),
    )(p2a, p2b, y2, dcol, b2)


def kernel(x, edge_index, W1, b1, W2, b2):
    n, _ = x.shape
    e = edge_index.shape[1]
    hid, ncls = W1.shape[1], W2.shape[1]
    src = edge_index[0]
    dst = edge_index[1]

    degp = _make_deg(n, e)(dst, jnp.zeros((n,), jnp.float32))
    dcol = lax.rsqrt(degp[0] + degp[1] + 1.0)[:, None]

    y1 = _tc_a(dcol, x, W1)
    p1 = _make_scatter(n, e, hid)(y1, src, dst, jnp.zeros((n, hid), jnp.float32))
    y2 = _tc_b(p1[0], p1[1], y1, dcol, b1, W2)
    p2 = _make_scatter(n, e, ncls)(y2, src, dst, jnp.zeros((n, ncls), jnp.float32))
    return _tc_c(p2[0], p2[1], y2, dcol, b2)


# speedup vs baseline: 51.0379x; 3.1769x over previous
import functools

import jax
import jax.numpy as jnp
from jax import lax
from jax.experimental import pallas as pl
from jax.experimental.pallas import tpu as pltpu
from jax.experimental.pallas import tpu_sc as plsc

NC, NS = 2, 16
NW = NC * NS
K = 80


def _sc_mesh():
    return plsc.VectorSubcoreMesh(
        core_axis_name="c", subcore_axis_name="s", num_cores=NC, num_subcores=NS)


def _make_deg(n, e, k=80):
    ept = e // NW
    ch = ept // k
    assert e % NW == 0 and ept % k == 0 and k % 16 == 0 and k <= 128

    def body(dst_hbm, zero_hbm, out_hbm, didx, ones, acc, sem):
        c = lax.axis_index("c")
        s = lax.axis_index("s")
        wid = c * NS + s

        @pl.when(s == 0)
        def _():
            pltpu.sync_copy(zero_hbm, acc)

        pltpu.sync_copy(dst_hbm.at[wid], didx)
        for j in range(k // 16):
            ones[pl.ds(j * 16, 16)] = jnp.full((16,), 1.0, jnp.float32)
        plsc.subcore_barrier()

        grp = 5
        assert ch % grp == 0
        def step(i, carry):
            for b in range(grp):
                pltpu.async_copy(ones, acc.at[didx.at[i * grp + b]], sem, add=True)
            for b in range(grp):
                pltpu.make_async_copy(ones, acc.at[didx.at[0]], sem).wait()
            return carry

        lax.fori_loop(0, ch // grp, step, 0)
        plsc.subcore_barrier()

        @pl.when(s == 0)
        def _():
            pltpu.sync_copy(acc, out_hbm.at[c])

    return pl.kernel(
        body,
        out_type=jax.ShapeDtypeStruct((NC, n), jnp.float32),
        mesh=_sc_mesh(),
        compiler_params=pltpu.CompilerParams(use_tc_tiling_on_sc=False),
        scratch_types=[
            pltpu.VMEM((ch, k), jnp.int32),
            pltpu.VMEM((k,), jnp.float32),
            pltpu.VMEM_SHARED((n,), jnp.float32),
            pltpu.SemaphoreType.DMA,
        ],
    ), ch, k


_NBUF = 4


def _make_scatter(n, e, f, k=125):
    ept = e // NW
    ch = ept // k
    assert e % NW == 0 and ept % k == 0 and k <= 128 and ch % _NBUF == 0

    def body(y_hbm, src_hbm, dst_hbm, zero_hbm, out_hbm,
             sidx, didx, rows, acc, gsem, ssem):
        c = lax.axis_index("c")
        s = lax.axis_index("s")
        wid = c * NS + s

        @pl.when(s == 0)
        def _():
            pltpu.sync_copy(zero_hbm, acc)

        pltpu.sync_copy(src_hbm.at[wid], sidx)
        pltpu.sync_copy(dst_hbm.at[wid], didx)
        plsc.subcore_barrier()

        def fire_gather(j, b):
            pltpu.async_copy(y_hbm.at[sidx.at[j]], rows.at[b], gsem.at[b])

        def wait_gather(j, b):
            pltpu.make_async_copy(
                y_hbm.at[sidx.at[j]], rows.at[b], gsem.at[b]).wait()

        def fire_scat(j, b):
            pltpu.async_copy(rows.at[b], acc.at[didx.at[j]], ssem.at[b],
                             add=True)

        def wait_scat(j, b):
            pltpu.make_async_copy(
                rows.at[b], acc.at[didx.at[j]], ssem.at[b]).wait()

        for b in range(_NBUF):
            fire_gather(b, b)

        def steady(io, carry):
            for b in range(_NBUF):
                j = io * _NBUF + b
                wait_gather(j, b)
                fire_scat(j, b)
            for b in range(_NBUF):
                j = io * _NBUF + b
                wait_scat(j, b)
                fire_gather(j + _NBUF, b)
            return carry

        lax.fori_loop(0, ch // _NBUF - 1, steady, 0)

        for b in range(_NBUF):
            j = ch - _NBUF + b
            wait_gather(j, b)
            fire_scat(j, b)
        for b in range(_NBUF):
            wait_scat(ch - _NBUF + b, b)
        plsc.subcore_barrier()

        @pl.when(s == 0)
        def _():
            pltpu.sync_copy(acc, out_hbm.at[c])

    return pl.kernel(
        body,
        out_type=jax.ShapeDtypeStruct((NC, n, f), jnp.float32),
        mesh=_sc_mesh(),
        compiler_params=pltpu.CompilerParams(use_tc_tiling_on_sc=False),
        scratch_types=[
            pltpu.VMEM((ch, k), jnp.int32),
            pltpu.VMEM((ch, k), jnp.int32),
            pltpu.VMEM((_NBUF, k, f), jnp.float32),
            pltpu.VMEM_SHARED((n, f), jnp.float32),
            pltpu.SemaphoreType.DMA((_NBUF,)),
            pltpu.SemaphoreType.DMA((_NBUF,)),
        ],
    ), ch, k


def _tc_a_body(dcol_ref, x_ref, w1_ref, y1_ref):
    xw = jnp.dot(x_ref[...], w1_ref[...], preferred_element_type=jnp.float32)
    y1_ref[...] = xw * dcol_ref[...]


def _tc_b_body(p1a_ref, p1b_ref, y1_ref, dcol_ref, b1_ref, w2_ref, y2_ref):
    d = dcol_ref[...]
    h = (p1a_ref[...] + p1b_ref[...] + y1_ref[...]) * d + b1_ref[...][None, :]
    h = jnp.maximum(h, 0.0)
    y2_ref[...] = jnp.dot(h, w2_ref[...], preferred_element_type=jnp.float32) * d


def _tc_c_body(p2a_ref, p2b_ref, y2_ref, dcol_ref, b2_ref, out_ref):
    o = (p2a_ref[...] + p2b_ref[...] + y2_ref[...]) * dcol_ref[...] \
        + b2_ref[...][None, :]
    m = jnp.max(o, axis=1, keepdims=True)
    ex = jnp.exp(o - m)
    out_ref[...] = o - m - jnp.log(jnp.sum(ex, axis=1, keepdims=True))


def _row_grid(n, r):
    assert n % r == 0
    return n // r


def _tc_a(dcol, x, w1, r=2000):
    n, dft = x.shape
    h = w1.shape[1]
    return pl.pallas_call(
        _tc_a_body,
        grid=(_row_grid(n, r),),
        in_specs=[
            pl.BlockSpec((r, 1), lambda g: (g, 0)),
            pl.BlockSpec((r, dft), lambda g: (g, 0)),
            pl.BlockSpec((dft, h), lambda g: (0, 0)),
        ],
        out_specs=pl.BlockSpec((r, h), lambda g: (g, 0)),
        out_shape=jax.ShapeDtypeStruct((n, h), jnp.float32),
    )(dcol, x, w1)


def _tc_b(p1a, p1b, y1, dcol, b1, w2, r=2000):
    n, h = y1.shape
    c = w2.shape[1]
    row = lambda g: (g, 0)
    return pl.pallas_call(
        _tc_b_body,
        grid=(_row_grid(n, r),),
        in_specs=[
            pl.BlockSpec((r, h), row),
            pl.BlockSpec((r, h), row),
            pl.BlockSpec((r, h), row),
            pl.BlockSpec((r, 1), row),
            pl.BlockSpec((h,), lambda g: (0,)),
            pl.BlockSpec((h, c), lambda g: (0, 0)),
        ],
        out_specs=pl.BlockSpec((r, c), row),
        out_shape=jax.ShapeDtypeStruct((n, c), jnp.float32),
    )(p1a, p1b, y1, dcol, b1, w2)


def _tc_c(p2a, p2b, y2, dcol, b2, r=2000):
    n, c = y2.shape
    row = lambda g: (g, 0)
    return pl.pallas_call(
        _tc_c_body,
        grid=(_row_grid(n, r),),
        in_specs=[
            pl.BlockSpec((r, c), row),
            pl.BlockSpec((r, c), row),
            pl.BlockSpec((r, c), row),
            pl.BlockSpec((r, 1), row),
            pl.BlockSpec((c,), lambda g: (0,)),
        ],
        out_specs=pl.BlockSpec((r, c), row),
        out_shape=jax.ShapeDtypeStruct((n, c), jnp.float32),
    )(p2a, p2b, y2, dcol, b2)


def kernel(x, edge_index, W1, b1, W2, b2):
    n, _ = x.shape
    e = edge_index.shape[1]
    hid, ncls = W1.shape[1], W2.shape[1]
    src = edge_index[0]
    dst = edge_index[1]

    deg_fn, dch, dk = _make_deg(n, e)
    degp = deg_fn(dst.reshape(NW, dch, dk), jnp.zeros((n,), jnp.float32))
    dcol = lax.rsqrt(degp[0] + degp[1] + 1.0)[:, None]

    s1_fn, ch, k = _make_scatter(n, e, hid)
    s2_fn, _, _ = _make_scatter(n, e, ncls)
    src3 = src.reshape(NW, ch, k)
    dst3 = dst.reshape(NW, ch, k)

    y1 = _tc_a(dcol, x, W1)
    p1 = s1_fn(y1, src3, dst3, jnp.zeros((n, hid), jnp.float32))
    y2 = _tc_b(p1[0], p1[1], y1, dcol, b1, W2)
    p2 = s2_fn(y2, src3, dst3, jnp.zeros((n, ncls), jnp.float32))
    return _tc_c(p2[0], p2[1], y2, dcol, b2)


# speedup vs baseline: 57.9756x; 1.1359x over previous
import functools

import jax
import jax.numpy as jnp
from jax import lax
from jax.experimental import pallas as pl
from jax.experimental.pallas import tpu as pltpu
from jax.experimental.pallas import tpu_sc as plsc

NC, NS = 2, 16
NW = NC * NS
K = 80


def _sc_mesh():
    return plsc.VectorSubcoreMesh(
        core_axis_name="c", subcore_axis_name="s", num_cores=NC, num_subcores=NS)


def _make_deg(n, e, k=80):
    ept = e // NW
    ch = ept // k
    assert e % NW == 0 and ept % k == 0 and k % 16 == 0 and k <= 128

    def body(ei_hbm, zero_hbm, out_hbm, didx, ones, acc, sem):
        c = lax.axis_index("c")
        s = lax.axis_index("s")
        wid = c * NS + s

        @pl.when(s == 0)
        def _():
            pltpu.sync_copy(zero_hbm, acc)

        pltpu.sync_copy(ei_hbm.at[1, wid], didx)
        for j in range(k // 16):
            ones[pl.ds(j * 16, 16)] = jnp.full((16,), 1.0, jnp.float32)
        plsc.subcore_barrier()

        grp = 5
        assert ch % grp == 0
        def step(i, carry):
            for b in range(grp):
                pltpu.async_copy(ones, acc.at[didx.at[i * grp + b]], sem, add=True)
            for b in range(grp):
                pltpu.make_async_copy(ones, acc.at[didx.at[0]], sem).wait()
            return carry

        lax.fori_loop(0, ch // grp, step, 0)
        plsc.subcore_barrier()

        @pl.when(s == 0)
        def _():
            pltpu.sync_copy(acc, out_hbm.at[c])

    return pl.kernel(
        body,
        out_type=jax.ShapeDtypeStruct((NC, n), jnp.float32),
        mesh=_sc_mesh(),
        compiler_params=pltpu.CompilerParams(use_tc_tiling_on_sc=False),
        scratch_types=[
            pltpu.VMEM((ch, k), jnp.int32),
            pltpu.VMEM((k,), jnp.float32),
            pltpu.VMEM_SHARED((n,), jnp.float32),
            pltpu.SemaphoreType.DMA,
        ],
    ), ch, k


_NBUF = 4


def _make_scatter(n, e, f, k=125):
    ept = e // NW
    ch = ept // k
    assert e % NW == 0 and ept % k == 0 and k <= 128 and ch % _NBUF == 0

    def body(y_hbm, ei_hbm, zero_hbm, out_hbm,
             sidx, didx, rows, acc, gsem, ssem):
        c = lax.axis_index("c")
        s = lax.axis_index("s")
        wid = c * NS + s

        @pl.when(s == 0)
        def _():
            pltpu.sync_copy(zero_hbm, acc)

        pltpu.sync_copy(ei_hbm.at[0, wid], sidx)
        pltpu.sync_copy(ei_hbm.at[1, wid], didx)
        plsc.subcore_barrier()

        def fire_gather(j, b):
            pltpu.async_copy(y_hbm.at[sidx.at[j]], rows.at[b], gsem.at[b])

        def wait_gather(j, b):
            pltpu.make_async_copy(
                y_hbm.at[sidx.at[j]], rows.at[b], gsem.at[b]).wait()

        def fire_scat(j, b):
            pltpu.async_copy(rows.at[b], acc.at[didx.at[j]], ssem.at[b],
                             add=True)

        def wait_scat(j, b):
            pltpu.make_async_copy(
                rows.at[b], acc.at[didx.at[j]], ssem.at[b]).wait()

        for b in range(_NBUF):
            fire_gather(b, b)

        def steady(io, carry):
            for b in range(_NBUF):
                j = io * _NBUF + b
                wait_gather(j, b)
                fire_scat(j, b)
            for b in range(_NBUF):
                j = io * _NBUF + b
                wait_scat(j, b)
                fire_gather(j + _NBUF, b)
            return carry

        lax.fori_loop(0, ch // _NBUF - 1, steady, 0)

        for b in range(_NBUF):
            j = ch - _NBUF + b
            wait_gather(j, b)
            fire_scat(j, b)
        for b in range(_NBUF):
            wait_scat(ch - _NBUF + b, b)
        plsc.subcore_barrier()

        @pl.when(s == 0)
        def _():
            pltpu.sync_copy(acc, out_hbm.at[c])

    return pl.kernel(
        body,
        out_type=jax.ShapeDtypeStruct((NC, n, f), jnp.float32),
        mesh=_sc_mesh(),
        compiler_params=pltpu.CompilerParams(use_tc_tiling_on_sc=False),
        scratch_types=[
            pltpu.VMEM((ch, k), jnp.int32),
            pltpu.VMEM((ch, k), jnp.int32),
            pltpu.VMEM((_NBUF, k, f), jnp.float32),
            pltpu.VMEM_SHARED((n, f), jnp.float32),
            pltpu.SemaphoreType.DMA((_NBUF,)),
            pltpu.SemaphoreType.DMA((_NBUF,)),
        ],
    ), ch, k


def _tc_a_body(dcol_ref, x_ref, w1_ref, y1_ref):
    xw = jnp.dot(x_ref[...], w1_ref[...], preferred_element_type=jnp.float32)
    y1_ref[...] = xw * dcol_ref[...]


def _tc_b_body(p1_ref, y1_ref, dcol_ref, b1_ref, w2_ref, y2_ref):
    d = dcol_ref[...]
    h = (p1_ref[0] + p1_ref[1] + y1_ref[...]) * d + b1_ref[...][None, :]
    h = jnp.maximum(h, 0.0)
    y2_ref[...] = jnp.dot(h, w2_ref[...], preferred_element_type=jnp.float32) * d


def _tc_c_body(p2_ref, y2_ref, dcol_ref, b2_ref, out_ref):
    o = (p2_ref[0] + p2_ref[1] + y2_ref[...]) * dcol_ref[...] \
        + b2_ref[...][None, :]
    m = jnp.max(o, axis=1, keepdims=True)
    ex = jnp.exp(o - m)
    out_ref[...] = o - m - jnp.log(jnp.sum(ex, axis=1, keepdims=True))


def _row_grid(n, r):
    assert n % r == 0
    return n // r


def _tc_a(dcol, x, w1, r=2000):
    n, dft = x.shape
    h = w1.shape[1]
    return pl.pallas_call(
        _tc_a_body,
        grid=(_row_grid(n, r),),
        in_specs=[
            pl.BlockSpec((r, 1), lambda g: (g, 0)),
            pl.BlockSpec((r, dft), lambda g: (g, 0)),
            pl.BlockSpec((dft, h), lambda g: (0, 0)),
        ],
        out_specs=pl.BlockSpec((r, h), lambda g: (g, 0)),
        out_shape=jax.ShapeDtypeStruct((n, h), jnp.float32),
    )(dcol, x, w1)


def _tc_b(p1, y1, dcol, b1, w2, r=2000):
    n, h = y1.shape
    c = w2.shape[1]
    row = lambda g: (g, 0)
    return pl.pallas_call(
        _tc_b_body,
        grid=(_row_grid(n, r),),
        in_specs=[
            pl.BlockSpec((2, r, h), lambda g: (0, g, 0)),
            pl.BlockSpec((r, h), row),
            pl.BlockSpec((r, 1), row),
            pl.BlockSpec((h,), lambda g: (0,)),
            pl.BlockSpec((h, c), lambda g: (0, 0)),
        ],
        out_specs=pl.BlockSpec((r, c), row),
        out_shape=jax.ShapeDtypeStruct((n, c), jnp.float32),
    )(p1, y1, dcol, b1, w2)


def _tc_c(p2, y2, dcol, b2, r=2000):
    n, c = y2.shape
    row = lambda g: (g, 0)
    return pl.pallas_call(
        _tc_c_body,
        grid=(_row_grid(n, r),),
        in_specs=[
            pl.BlockSpec((2, r, c), lambda g: (0, g, 0)),
            pl.BlockSpec((r, c), row),
            pl.BlockSpec((r, 1), row),
            pl.BlockSpec((c,), lambda g: (0,)),
        ],
        out_specs=pl.BlockSpec((r, c), row),
        out_shape=jax.ShapeDtypeStruct((n, c), jnp.float32),
    )(p2, y2, dcol, b2)


def kernel(x, edge_index, W1, b1, W2, b2):
    n, _ = x.shape
    e = edge_index.shape[1]
    hid, ncls = W1.shape[1], W2.shape[1]
    deg_fn, dch, dk = _make_deg(n, e)
    degp = deg_fn(edge_index.reshape(2, NW, dch, dk),
                  jnp.zeros((n,), jnp.float32))
    dcol = lax.rsqrt(degp[0] + degp[1] + 1.0)[:, None]

    s1_fn, ch, k = _make_scatter(n, e, hid)
    s2_fn, _, _ = _make_scatter(n, e, ncls)
    ei4 = edge_index.reshape(2, NW, ch, k)

    y1 = _tc_a(dcol, x, W1)
    p1 = s1_fn(y1, ei4, jnp.zeros((n, hid), jnp.float32))
    y2 = _tc_b(p1, y1, dcol, b1, W2)
    p2 = s2_fn(y2, ei4, jnp.zeros((n, ncls), jnp.float32))
    return _tc_c(p2, y2, dcol, b2)


# speedup vs baseline: 93.2044x; 1.6076x over previous
import functools

import jax
import jax.numpy as jnp
from jax import lax
from jax.experimental import pallas as pl
from jax.experimental.pallas import tpu as pltpu
from jax.experimental.pallas import tpu_sc as plsc

NC, NS = 2, 16
NW = NC * NS
K = 80


def _sc_mesh():
    return plsc.VectorSubcoreMesh(
        core_axis_name="c", subcore_axis_name="s", num_cores=NC, num_subcores=NS)


def _make_deg(n, e, k=125):
    ept = e // NW
    ch = ept // k
    kpad = ((k + 15) // 16) * 16
    assert e % NW == 0 and ept % k == 0 and k <= 128

    def body(ei_hbm, zero_hbm, out_hbm, didx, ones, acc, sem):
        c = lax.axis_index("c")
        s = lax.axis_index("s")
        wid = c * NS + s

        @pl.when(s == 0)
        def _():
            pltpu.sync_copy(zero_hbm, acc)

        pltpu.sync_copy(ei_hbm.at[1, wid], didx)
        for j in range(kpad // 16):
            ones[pl.ds(j * 16, 16)] = jnp.full((16,), 1.0, jnp.float32)
        plsc.subcore_barrier()

        grp = 10
        assert ch % grp == 0
        def step(i, carry):
            for b in range(grp):
                pltpu.async_copy(ones.at[pl.ds(0, k)],
                                 acc.at[didx.at[i * grp + b]], sem, add=True)
            for b in range(grp):
                pltpu.make_async_copy(ones.at[pl.ds(0, k)],
                                      acc.at[didx.at[0]], sem).wait()
            return carry

        lax.fori_loop(0, ch // grp, step, 0)
        plsc.subcore_barrier()

        @pl.when(s == 0)
        def _():
            pltpu.sync_copy(acc, out_hbm.at[c])

    return pl.kernel(
        body,
        out_type=jax.ShapeDtypeStruct((NC, n), jnp.float32),
        mesh=_sc_mesh(),
        compiler_params=pltpu.CompilerParams(use_tc_tiling_on_sc=False),
        scratch_types=[
            pltpu.VMEM((ch, k), jnp.int32),
            pltpu.VMEM((kpad,), jnp.float32),
            pltpu.VMEM_SHARED((n,), jnp.float32),
            pltpu.SemaphoreType.DMA,
        ],
    ), ch, k


_NBUF = 8


def _make_scatter(n, e, f, k=125):
    ept = e // NW
    ch = ept // k
    assert e % NW == 0 and ept % k == 0 and k <= 128 and ch % _NBUF == 0

    def body(y_hbm, ei_hbm, zero_hbm, out_hbm,
             sidx, didx, rows, acc, gsem, ssem):
        c = lax.axis_index("c")
        s = lax.axis_index("s")
        wid = c * NS + s

        @pl.when(s == 0)
        def _():
            pltpu.sync_copy(zero_hbm, acc)

        pltpu.sync_copy(ei_hbm.at[0, wid], sidx)
        pltpu.sync_copy(ei_hbm.at[1, wid], didx)
        plsc.subcore_barrier()

        def fire_gather(j, b):
            pltpu.async_copy(y_hbm.at[sidx.at[j]], rows.at[b], gsem.at[b])

        def wait_gather(j, b):
            pltpu.make_async_copy(
                y_hbm.at[sidx.at[j]], rows.at[b], gsem.at[b]).wait()

        def fire_scat(j, b):
            pltpu.async_copy(rows.at[b], acc.at[didx.at[j]], ssem.at[b],
                             add=True)

        def wait_scat(j, b):
            pltpu.make_async_copy(
                rows.at[b], acc.at[didx.at[j]], ssem.at[b]).wait()

        for b in range(_NBUF):
            fire_gather(b, b)

        def steady(io, carry):
            for b in range(_NBUF):
                j = io * _NBUF + b
                wait_gather(j, b)
                fire_scat(j, b)
            for b in range(_NBUF):
                j = io * _NBUF + b
                wait_scat(j, b)
                fire_gather(j + _NBUF, b)
            return carry

        lax.fori_loop(0, ch // _NBUF - 1, steady, 0)

        for b in range(_NBUF):
            j = ch - _NBUF + b
            wait_gather(j, b)
            fire_scat(j, b)
        for b in range(_NBUF):
            wait_scat(ch - _NBUF + b, b)
        plsc.subcore_barrier()

        @pl.when(s == 0)
        def _():
            pltpu.sync_copy(acc, out_hbm.at[c])

    return pl.kernel(
        body,
        out_type=jax.ShapeDtypeStruct((NC, n, f), jnp.float32),
        mesh=_sc_mesh(),
        compiler_params=pltpu.CompilerParams(use_tc_tiling_on_sc=False),
        scratch_types=[
            pltpu.VMEM((ch, k), jnp.int32),
            pltpu.VMEM((ch, k), jnp.int32),
            pltpu.VMEM((_NBUF, k, f), jnp.float32),
            pltpu.VMEM_SHARED((n, f), jnp.float32),
            pltpu.SemaphoreType.DMA((_NBUF,)),
            pltpu.SemaphoreType.DMA((_NBUF,)),
        ],
    ), ch, k


def _tc_a_body(x_ref, w1bd_ref, xw_ref):
    xw_ref[...] = jnp.dot(x_ref[...], w1bd_ref[...],
                          preferred_element_type=jnp.float32)


def _tc_b_body(p1_ref, y1_ref, dflat_ref, b1f_ref, g_ref):
    d = dflat_ref[...]
    h = (p1_ref[0] + p1_ref[1] + y1_ref[...]) * d + b1f_ref[...][None, :]
    g_ref[...] = jnp.maximum(h, 0.0) * d


def _tc_c_body(p2_ref, g_ref, dflat_ref, w2e_ref, b2_ref, out_ref):
    mf = (p2_ref[0] + p2_ref[1] + g_ref[...]) * dflat_ref[...]
    for j in range(8):
        o = jnp.dot(mf, w2e_ref[j], preferred_element_type=jnp.float32) \
            + b2_ref[...][None, :]
        m = jnp.max(o, axis=1, keepdims=True)
        ex = jnp.exp(o - m)
        out_ref[:, j, :] = o - m - jnp.log(jnp.sum(ex, axis=1, keepdims=True))


def _tc_a(xr, w1bd):
    nf = xr.shape[0]
    return pl.pallas_call(
        _tc_a_body,
        out_shape=jax.ShapeDtypeStruct((nf, 128), jnp.float32),
    )(xr, w1bd)


def _tc_b(p1f, y1f, dflat, b1f):
    nf = y1f.shape[0]
    return pl.pallas_call(
        _tc_b_body,
        out_shape=jax.ShapeDtypeStruct((nf, 128), jnp.float32),
    )(p1f, y1f, dflat, b1f)


def _tc_c(p2f, gf, dflat, w2e, b2):
    nf = gf.shape[0]
    c = b2.shape[0]
    return pl.pallas_call(
        _tc_c_body,
        out_shape=jax.ShapeDtypeStruct((nf, 8, c), jnp.float32),
    )(p2f, gf, dflat, w2e, b2)


def kernel(x, edge_index, W1, b1, W2, b2):
    n, dft = x.shape
    e = edge_index.shape[1]
    hid, ncls = W1.shape[1], W2.shape[1]
    assert hid == 16 and n % 8 == 0
    nf = n * hid // 128

    s_fn, ch, k = _make_scatter(n, e, hid)
    deg_fn, dch, dk = _make_deg(n, e)
    assert (dch, dk) == (ch, k)
    ei4 = edge_index.reshape(2, NW, ch, k)

    degp = deg_fn(ei4, jnp.zeros((n,), jnp.float32))
    dvec = lax.rsqrt(degp[0] + degp[1] + 1.0)
    dflat = jnp.repeat(dvec, hid).reshape(nf, 128)

    w1bd = jax.scipy.linalg.block_diag(*([W1] * 8))
    b1f = jnp.tile(b1, 8)
    w2e = jnp.stack([jnp.pad(W2, ((hid * j, 128 - hid * (j + 1)), (0, 0)))
                     for j in range(8)])

    zf = jnp.zeros((n, hid), jnp.float32)

    xwf = _tc_a(x.reshape(nf, 8 * dft), w1bd)
    y1f = xwf * dflat
    p1 = s_fn(y1f.reshape(n, hid), ei4, zf)
    gf = _tc_b(p1.reshape(2, nf, 128), y1f, dflat, b1f)
    p2 = s_fn(gf.reshape(n, hid), ei4, zf)
    out = _tc_c(p2.reshape(2, nf, 128), gf, dflat, w2e, b2)
    return out.reshape(n, ncls)
